# baseline (speedup 1.0000x reference)
import jax
import jax.numpy as jnp
from jax.experimental import pallas as pl
from jax.experimental.pallas import tpu as pltpu

_EPS = 1e-12
_GB = 2
_STEPS = 4


def _vq_kernel(z_ref, emb_ref, zq_ref, idx_ref, loss_ref,
               idxcol_ref, enbf_ref, enm2_ref, col_ref, tot_ref):
    step = pl.program_id(0)

    @pl.when(step == 0)
    def _prep():
        e = emb_ref[...]
        en = e * jax.lax.rsqrt(jnp.sum(e * e, axis=1, keepdims=True) + _EPS)
        col_ref[...] = jnp.sum(en * en, axis=1)[None, :]
        enbf_ref[...] = en.astype(jnp.bfloat16)
        enm2_ref[...] = en * jnp.float32(-2.0)
        tot_ref[0] = jnp.float32(0.0)

    en_bf = enbf_ref[...]
    en_m2 = enm2_ref[...]
    colterm = col_ref[0, :]

    total = tot_ref[0]
    for bi in range(_GB):
        z = z_ref[bi]
        zn = z * jax.lax.rsqrt(jnp.sum(z * z, axis=1, keepdims=True) + _EPS)
        rowterm = jnp.sum(zn * zn, axis=1, keepdims=True)
        dots_m2 = jax.lax.dot_general(
            zn, en_m2, (((1,), (1,)), ((), ())),
            preferred_element_type=jnp.float32)
        d = (rowterm + colterm) + dots_m2
        minval = jnp.min(d, axis=1, keepdims=True)
        lanes = jax.lax.broadcasted_iota(
            jnp.int32, d.shape, 1).astype(jnp.float32)
        idx_f = jnp.min(jnp.where(d == minval, lanes, jnp.float32(2**30)),
                        axis=1)
        for k in range(_STEPS):
            @pl.when(step == k)
            def _(k=k, idx_f=idx_f, bi=bi):
                idxcol_ref[:, k * _GB + bi] = idx_f
        total += jnp.sum(minval)
        onehot = (lanes == idx_f[:, None]).astype(jnp.bfloat16)
        zq_ref[bi] = jax.lax.dot_general(
            onehot, en_bf, (((1,), (0,)), ((), ())),
            preferred_element_type=jnp.float32)
    tot_ref[0] = total

    @pl.when(step == _STEPS - 1)
    def _fin():
        idx_ref[...] = idxcol_ref[...].T.astype(jnp.int32)
        n = _STEPS * _GB * z_ref.shape[1] * z_ref.shape[2]
        m = tot_ref[0] / n
        loss_ref[0, 0] = jnp.float32(0.25) * m + m


def kernel(z, embedding):
    b, t, c = z.shape

    zq, idx, loss = pl.pallas_call(
        _vq_kernel,
        grid=(_STEPS,),
        in_specs=[
            pl.BlockSpec((_GB, t, c), lambda i: (i, 0, 0)),
            pl.BlockSpec(embedding.shape, lambda i: (0, 0)),
        ],
        out_specs=[
            pl.BlockSpec((_GB, t, c), lambda i: (i, 0, 0)),
            pl.BlockSpec((_STEPS * _GB, t), lambda i: (0, 0)),
            pl.BlockSpec(memory_space=pltpu.SMEM),
        ],
        out_shape=[
            jax.ShapeDtypeStruct(z.shape, jnp.float32),
            jax.ShapeDtypeStruct((b, t), jnp.int32),
            jax.ShapeDtypeStruct((1, 1), jnp.float32),
        ],
        scratch_shapes=[
            pltpu.VMEM((t, _STEPS * _GB), jnp.float32),
            pltpu.VMEM(embedding.shape, jnp.bfloat16),
            pltpu.VMEM(embedding.shape, jnp.float32),
            pltpu.VMEM((1, embedding.shape[0]), jnp.float32),
            pltpu.SMEM((1,), jnp.float32),
        ],
    )(z, embedding)

    return (zq, loss[0, 0], idx)


# speedup vs baseline: 1.3148x; 1.3148x over previous
import jax
import jax.numpy as jnp
from jax.experimental import pallas as pl
from jax.experimental.pallas import tpu as pltpu

_EPS = 1e-12


def _vq_kernel(z_ref, emb_ref, zq_ref, idx_ref, loss_ref, idxcol_ref):
    e = emb_ref[...]
    en = e * jax.lax.rsqrt(jnp.sum(e * e, axis=1, keepdims=True) + _EPS)
    colterm = jnp.sum(en * en, axis=1)
    en_bf = en.astype(jnp.bfloat16)
    en_m2 = en * jnp.float32(-2.0)

    total = jnp.float32(0.0)
    for bi in range(z_ref.shape[0]):
        z = z_ref[bi]
        zn = z * jax.lax.rsqrt(jnp.sum(z * z, axis=1, keepdims=True) + _EPS)
        rowterm = jnp.sum(zn * zn, axis=1, keepdims=True)
        dots_m2 = jax.lax.dot_general(
            zn, en_m2, (((1,), (1,)), ((), ())),
            preferred_element_type=jnp.float32)
        d = (rowterm + colterm) + dots_m2
        minval = jnp.min(d, axis=1, keepdims=True)
        lanes = jax.lax.broadcasted_iota(
            jnp.int32, d.shape, 1).astype(jnp.float32)
        idx_f = jnp.min(jnp.where(d == minval, lanes, jnp.float32(2**30)),
                        axis=1)
        idxcol_ref[:, bi] = idx_f
        total += jnp.sum(minval)
        onehot = (lanes == idx_f[:, None]).astype(jnp.bfloat16)
        zq_ref[bi] = jax.lax.dot_general(
            onehot, en_bf, (((1,), (0,)), ((), ())),
            preferred_element_type=jnp.float32)

    idx_ref[...] = idxcol_ref[...].T.astype(jnp.int32)

    n = z_ref.shape[0] * z_ref.shape[1] * z_ref.shape[2]
    m = total / n
    loss_ref[0, 0] = jnp.float32(0.25) * m + m


def kernel(z, embedding):
    b, t, c = z.shape

    zq, idx, loss = pl.pallas_call(
        _vq_kernel,
        in_specs=[
            pl.BlockSpec(z.shape, lambda: (0, 0, 0)),
            pl.BlockSpec(embedding.shape, lambda: (0, 0)),
        ],
        out_specs=[
            pl.BlockSpec(z.shape, lambda: (0, 0, 0)),
            pl.BlockSpec((b, t), lambda: (0, 0)),
            pl.BlockSpec(memory_space=pltpu.SMEM),
        ],
        out_shape=[
            jax.ShapeDtypeStruct(z.shape, jnp.float32),
            jax.ShapeDtypeStruct((b, t), jnp.int32),
            jax.ShapeDtypeStruct((1, 1), jnp.float32),
        ],
        scratch_shapes=[pltpu.VMEM((t, b), jnp.float32)],
    )(z, embedding)

    return (zq, loss[0, 0], idx)
